# Initial kernel scaffold; baseline (speedup 1.0000x reference)
import jax
import jax.numpy as jnp
from jax.experimental import pallas as pl


def kernel(u, v, table, salts):
    raise NotImplementedError("write your pallas kernel here")


# speedup vs baseline: 2087.1722x; 2087.1722x over previous
import functools

import jax
import jax.numpy as jnp
from jax import lax
from jax.experimental import pallas as pl
from jax.experimental.pallas import tpu as pltpu
from jax.experimental.pallas import tpu_sc as plsc

_C = 10000
_NS = 16
_NC = 2

_MUL_U = 31337
_MUL_V = -1640531527


def _sc_sketch(u32, v32, table, salts16):
    n = u32.shape[0]
    depth, width = table.shape
    mask = jnp.int32(width - 1)
    per_tile = n // _NS
    n_chunks = per_tile // _C
    seg = width // _NS
    mesh = plsc.VectorSubcoreMesh(core_axis_name="c", subcore_axis_name="s")

    @functools.partial(
        pl.kernel,
        out_type=jax.ShapeDtypeStruct((_NC * n, ), jnp.float32),
        mesh=mesh,
        scratch_types=[
            pltpu.VMEM_SHARED((width,), jnp.float32),
            pltpu.VMEM((_C,), jnp.int32),
            pltpu.VMEM((_C,), jnp.int32),
            pltpu.VMEM((_C,), jnp.int32),
            pltpu.VMEM((_C,), jnp.float32),
            pltpu.VMEM((_C,), jnp.float32),
            pltpu.VMEM((_C,), jnp.float32),
            pltpu.VMEM((16,), jnp.int32),
        ],
    )
    def body(u_hbm, v_hbm, tab_hbm, salt_hbm, part_hbm,
             row_sh, u_v, v_v, idx_v, ones_v, vals_v, prev_v, salt_v):
        c = lax.axis_index("c")
        s = lax.axis_index("s")
        tile_base = s * jnp.int32(per_tile)

        def fill_ones(i, carry):
            idx16 = pl.multiple_of(i * jnp.int32(16), 16)
            ones_v[pl.ds(idx16, 16)] = jnp.full((16,), 1.0, jnp.float32)
            return carry

        lax.fori_loop(jnp.int32(0), jnp.int32(_C // 16), fill_ones, jnp.int32(0))

        for r in range(2):
            d = c * jnp.int32(2) + jnp.int32(r)
            pltpu.sync_copy(tab_hbm.at[d, pl.ds(s * jnp.int32(seg), seg)],
                            row_sh.at[pl.ds(s * jnp.int32(seg), seg)])
            pltpu.sync_copy(salt_hbm.at[d], salt_v)
            plsc.subcore_barrier()
            salt = salt_v[...]

            def hash_chunk():
                def hb(i, carry):
                    o = pl.multiple_of(i * jnp.int32(16), 16)
                    cb = (u_v[pl.ds(o, 16)] * jnp.int32(_MUL_U)
                          + v_v[pl.ds(o, 16)] * jnp.int32(_MUL_V))
                    idx_v[pl.ds(o, 16)] = (cb + salt) & mask
                    return carry

                lax.fori_loop(jnp.int32(0), jnp.int32(_C // 16), hb, jnp.int32(0))

            def upd_chunk(i, carry):
                base = pl.multiple_of(tile_base + i * jnp.int32(_C), 8)
                pltpu.sync_copy(u_hbm.at[pl.ds(base, _C)], u_v)
                pltpu.sync_copy(v_hbm.at[pl.ds(base, _C)], v_v)
                hash_chunk()
                pltpu.sync_copy(ones_v, row_sh.at[idx_v], add=True)
                return carry

            lax.fori_loop(jnp.int32(0), jnp.int32(n_chunks), upd_chunk, jnp.int32(0))
            plsc.subcore_barrier()

            def qry_chunk(i, carry):
                base = pl.multiple_of(tile_base + i * jnp.int32(_C), 8)
                pltpu.sync_copy(u_hbm.at[pl.ds(base, _C)], u_v)
                pltpu.sync_copy(v_hbm.at[pl.ds(base, _C)], v_v)
                hash_chunk()
                pltpu.sync_copy(row_sh.at[idx_v], vals_v)
                obase = pl.multiple_of(c * jnp.int32(n) + base, 8)
                if r == 1:
                    pltpu.sync_copy(part_hbm.at[pl.ds(obase, _C)], prev_v)

                    def mb(j, carry2):
                        o = pl.multiple_of(j * jnp.int32(16), 16)
                        vals_v[pl.ds(o, 16)] = jnp.minimum(
                            vals_v[pl.ds(o, 16)], prev_v[pl.ds(o, 16)])
                        return carry2

                    lax.fori_loop(jnp.int32(0), jnp.int32(_C // 16), mb, jnp.int32(0))
                pltpu.sync_copy(vals_v, part_hbm.at[pl.ds(obase, _C)])
                return carry

            lax.fori_loop(jnp.int32(0), jnp.int32(n_chunks), qry_chunk, jnp.int32(0))
            plsc.subcore_barrier()

    return body(u32, v32, table, salts16)


def _min2_kernel(p_ref, o_ref):
    o_ref[...] = jnp.minimum(p_ref[0], p_ref[1])


def kernel(u, v, table, salts):
    n = u.shape[0]
    u32 = u.astype(jnp.int32)
    v32 = v.astype(jnp.int32)
    salts16 = jnp.broadcast_to(salts.astype(jnp.int32), (table.shape[0], 16))
    part = _sc_sketch(u32, v32, table.astype(jnp.float32), salts16)

    rows = n // (50 * 128)
    grid = 25
    blk = rows // grid
    out = pl.pallas_call(
        _min2_kernel,
        grid=(grid,),
        in_specs=[pl.BlockSpec((_NC, blk, 50, 128), lambda i: (jnp.int32(0), i, jnp.int32(0), jnp.int32(0)))],
        out_specs=pl.BlockSpec((blk, 50, 128), lambda i: (i, jnp.int32(0), jnp.int32(0))),
        out_shape=jax.ShapeDtypeStruct((rows, 50, 128), jnp.float32),
    )(part.reshape(_NC, rows, 50, 128))
    return out.reshape(n)


# speedup vs baseline: 3485.8364x; 1.6701x over previous
import functools

import jax
import jax.numpy as jnp
from jax import lax
from jax.experimental import pallas as pl
from jax.experimental.pallas import tpu as pltpu
from jax.experimental.pallas import tpu_sc as plsc

_C = 2000
_NS = 16
_NC = 2
_UN = 5

_MUL_U = 31337
_MUL_V = -1640531527


def _i32(x):
    return jnp.int32(x)


def _sc_sketch(u32, v32, table, salts16):
    n = u32.shape[0]
    depth, width = table.shape
    mask = _i32(width - 1)
    per_tile = n // _NS
    nch = per_tile // _C
    npair = nch // 2
    seg = width // _NS
    nvec = _C // 16
    mesh = plsc.VectorSubcoreMesh(core_axis_name="c", subcore_axis_name="s")

    @functools.partial(
        pl.kernel,
        out_type=(jax.ShapeDtypeStruct((n,), jnp.float32),
                  jax.ShapeDtypeStruct((n,), jnp.float32)),
        mesh=mesh,
        scratch_types=[
            pltpu.VMEM_SHARED((width,), jnp.float32),
            pltpu.VMEM((_C,), jnp.int32),
            pltpu.VMEM((_C,), jnp.int32),
            pltpu.VMEM((_C,), jnp.int32),
            pltpu.VMEM((_C,), jnp.int32),
            pltpu.VMEM((_C,), jnp.int32),
            pltpu.VMEM((_C,), jnp.int32),
            pltpu.VMEM((_C,), jnp.float32),
            pltpu.VMEM((_C,), jnp.float32),
            pltpu.VMEM((_C,), jnp.float32),
            pltpu.VMEM((_C,), jnp.float32),
            pltpu.VMEM((16,), jnp.int32),
            pltpu.SemaphoreType.DMA,
            pltpu.SemaphoreType.DMA,
            pltpu.SemaphoreType.DMA,
            pltpu.SemaphoreType.DMA,
            pltpu.SemaphoreType.DMA,
            pltpu.SemaphoreType.DMA,
            pltpu.SemaphoreType.DMA,
            pltpu.SemaphoreType.DMA,
            pltpu.SemaphoreType.DMA,
            pltpu.SemaphoreType.DMA,
            pltpu.SemaphoreType.DMA,
        ],
    )
    def body(u_hbm, v_hbm, tab_hbm, salt_hbm, part0_hbm, part1_hbm,
             row_sh, u0, u1, v0, v1, i0, i1, ones_v, g0, g1, prev_v, salt_v,
             siu0, siu1, siv0, siv1, ssc0, ssc1, sg0, sg1, sprev,
             so0, so1):
        c = lax.axis_index("c")
        s = lax.axis_index("s")
        tile_base = s * _i32(per_tile)
        u_b, v_b, idx_b = (u0, u1), (v0, v1), (i0, i1)
        vals_b = (g0, g1)
        siu, siv = (siu0, siu1), (siv0, siv1)
        ssc, sg, so = (ssc0, ssc1), (sg0, sg1), (so0, so1)

        def fo(k, carry):
            for t in range(_UN):
                o = pl.multiple_of(k * _i32(16 * _UN) + _i32(t * 16), 16)
                ones_v[pl.ds(o, 16)] = jnp.full((16,), 1.0, jnp.float32)
            return carry

        lax.fori_loop(_i32(0), _i32(nvec // _UN), fo, _i32(0))

        def in_start(i, b):
            base = pl.multiple_of(tile_base + i * _i32(_C), 8)
            pltpu.async_copy(u_hbm.at[pl.ds(base, _C)], u_b[b], siu[b])
            pltpu.async_copy(v_hbm.at[pl.ds(base, _C)], v_b[b], siv[b])

        def in_wait(b):
            z = pl.ds(_i32(0), _C)
            pltpu.make_async_copy(u_hbm.at[z], u_b[b], siu[b]).wait()
            pltpu.make_async_copy(v_hbm.at[z], v_b[b], siv[b]).wait()

        def sc_wait(b):
            pltpu.make_async_copy(ones_v, row_sh.at[idx_b[b]],
                                  ssc[b]).wait()

        def out_wait(b):
            z = pl.ds(_i32(0), _C)
            pltpu.make_async_copy(vals_b[b], part0_hbm.at[z], so[b]).wait()

        for r in range(2):
            d = c * _i32(2) + _i32(r)
            pltpu.sync_copy(tab_hbm.at[d, pl.ds(s * _i32(seg), seg)],
                            row_sh.at[pl.ds(s * _i32(seg), seg)])
            pltpu.sync_copy(salt_hbm.at[d], salt_v)
            plsc.subcore_barrier()
            salt = salt_v[...]

            def hash_into(b):
                ub, vb, ib = u_b[b], v_b[b], idx_b[b]

                def hb(k, carry):
                    for t in range(_UN):
                        o = pl.multiple_of(
                            k * _i32(16 * _UN) + _i32(t * 16), 16)
                        sl = pl.ds(o, 16)
                        cb = (ub[sl] * _i32(_MUL_U)
                              + vb[sl] * _i32(_MUL_V))
                        ib[sl] = (cb + salt) & mask
                    return carry

                lax.fori_loop(_i32(0), _i32(nvec // _UN), hb, _i32(0))

            def upd_chunk(i, b, start_next):
                if start_next:
                    in_start(i + _i32(1), 1 - b)
                in_wait(b)
                hash_into(b)
                pltpu.async_copy(ones_v, row_sh.at[idx_b[b]], ssc[b],
                                 add=True)

            in_start(_i32(0), 0)

            def upd_pair(gp, carry):
                for b in (0, 1):
                    i = gp * _i32(2) + _i32(b)

                    @pl.when(gp >= _i32(1))
                    def _():
                        sc_wait(b)

                    upd_chunk(i, b, True)
                return carry

            lax.fori_loop(_i32(0), _i32(npair), upd_pair, _i32(0))
            sc_wait(0)
            upd_chunk(_i32(nch - 1), 0, False)
            sc_wait(1)
            sc_wait(0)
            plsc.subcore_barrier()

            def qry_chunk(i, b, start_next):
                if start_next:
                    in_start(i + _i32(1), 1 - b)
                in_wait(b)
                hash_into(b)
                obase = pl.multiple_of(tile_base + i * _i32(_C), 8)
                osl = pl.ds(obase, _C)
                pltpu.async_copy(row_sh.at[idx_b[b]], vals_b[b], sg[b])
                if r == 1:
                    @pl.when(c == _i32(0))
                    def _():
                        pltpu.async_copy(part0_hbm.at[osl], prev_v, sprev)

                    @pl.when(c == _i32(1))
                    def _():
                        pltpu.async_copy(part1_hbm.at[osl], prev_v, sprev)
                pltpu.make_async_copy(row_sh.at[idx_b[b]], vals_b[b],
                                      sg[b]).wait()
                if r == 1:
                    pltpu.make_async_copy(part0_hbm.at[osl], prev_v,
                                          sprev).wait()
                    vb, pb = vals_b[b], prev_v

                    def mb(k, carry):
                        for t in range(_UN):
                            o = pl.multiple_of(
                                k * _i32(16 * _UN) + _i32(t * 16), 16)
                            sl = pl.ds(o, 16)
                            vb[sl] = jnp.minimum(vb[sl], pb[sl])
                        return carry

                    lax.fori_loop(_i32(0), _i32(nvec // _UN), mb, _i32(0))
                @pl.when(c == _i32(0))
                def _():
                    pltpu.async_copy(vals_b[b], part0_hbm.at[osl], so[b])

                @pl.when(c == _i32(1))
                def _():
                    pltpu.async_copy(vals_b[b], part1_hbm.at[osl], so[b])

            in_start(_i32(0), 0)

            def qry_pair(gp, carry):
                for b in (0, 1):
                    i = gp * _i32(2) + _i32(b)

                    @pl.when(gp >= _i32(1))
                    def _():
                        out_wait(b)

                    qry_chunk(i, b, True)
                return carry

            lax.fori_loop(_i32(0), _i32(npair), qry_pair, _i32(0))
            out_wait(0)
            qry_chunk(_i32(nch - 1), 0, False)
            out_wait(1)
            out_wait(0)
            plsc.subcore_barrier()

    return body(u32, v32, table, salts16)


def _min2_kernel(a_ref, b_ref, o_ref):
    o_ref[...] = jnp.minimum(a_ref[...], b_ref[...])


def kernel(u, v, table, salts):
    n = u.shape[0]
    u32 = u.astype(jnp.int32)
    v32 = v.astype(jnp.int32)
    salts16 = jnp.broadcast_to(salts.astype(jnp.int32), (table.shape[0], 16))
    part0, part1 = _sc_sketch(u32, v32, table.astype(jnp.float32), salts16)

    blk = 1 << 17
    grid = -(-n // blk)
    out = pl.pallas_call(
        _min2_kernel,
        grid=(grid,),
        in_specs=[
            pl.BlockSpec((blk,), lambda i: (i,)),
            pl.BlockSpec((blk,), lambda i: (i,)),
        ],
        out_specs=pl.BlockSpec((blk,), lambda i: (i,)),
        out_shape=jax.ShapeDtypeStruct((n,), jnp.float32),
    )(part0, part1)
    return out


# speedup vs baseline: 3489.5643x; 1.0011x over previous
import functools

import jax
import jax.numpy as jnp
from jax import lax
from jax.experimental import pallas as pl
from jax.experimental.pallas import tpu as pltpu
from jax.experimental.pallas import tpu_sc as plsc

_C = 2000
_NS = 16
_NC = 2
_UN = 5

_MUL_U = 31337
_MUL_V = -1640531527


def _i32(x):
    return jnp.int32(x)


def _sc_sketch(u32, v32, table, salts16):
    n = u32.shape[0]
    depth, width = table.shape
    mask = _i32(width - 1)
    per_tile = n // _NS
    nch = per_tile // _C
    npair = nch // 2
    seg = width // _NS
    nvec = _C // 16
    mesh = plsc.VectorSubcoreMesh(core_axis_name="c", subcore_axis_name="s")

    @functools.partial(
        pl.kernel,
        out_type=(jax.ShapeDtypeStruct((n,), jnp.float32),
                  jax.ShapeDtypeStruct((n,), jnp.float32)),
        mesh=mesh,
        scratch_types=[
            pltpu.VMEM_SHARED((width,), jnp.float32),
            pltpu.VMEM((_C,), jnp.int32),
            pltpu.VMEM((_C,), jnp.int32),
            pltpu.VMEM((_C,), jnp.int32),
            pltpu.VMEM((_C,), jnp.int32),
            pltpu.VMEM((_C,), jnp.int32),
            pltpu.VMEM((_C,), jnp.int32),
            pltpu.VMEM((_C,), jnp.float32),
            pltpu.VMEM((_C,), jnp.float32),
            pltpu.VMEM((_C,), jnp.float32),
            pltpu.VMEM((_C,), jnp.float32),
            pltpu.VMEM((16,), jnp.int32),
            pltpu.SemaphoreType.DMA,
            pltpu.SemaphoreType.DMA,
            pltpu.SemaphoreType.DMA,
            pltpu.SemaphoreType.DMA,
            pltpu.SemaphoreType.DMA,
            pltpu.SemaphoreType.DMA,
            pltpu.SemaphoreType.DMA,
            pltpu.SemaphoreType.DMA,
            pltpu.SemaphoreType.DMA,
            pltpu.SemaphoreType.DMA,
            pltpu.SemaphoreType.DMA,
        ],
    )
    def body(u_hbm, v_hbm, tab_hbm, salt_hbm, part0_hbm, part1_hbm,
             row_sh, u0, u1, v0, v1, i0, i1, ones_v, g0, g1, prev_v, salt_v,
             siu0, siu1, siv0, siv1, ssc0, ssc1, sg0, sg1, sprev,
             so0, so1):
        c = lax.axis_index("c")
        s = lax.axis_index("s")
        tile_base = s * _i32(per_tile)
        u_b, v_b, idx_b = (u0, u1), (v0, v1), (i0, i1)
        vals_b = (g0, g1)
        siu, siv = (siu0, siu1), (siv0, siv1)
        ssc, sg, so = (ssc0, ssc1), (sg0, sg1), (so0, so1)

        def fo(k, carry):
            for t in range(_UN):
                o = pl.multiple_of(k * _i32(16 * _UN) + _i32(t * 16), 16)
                ones_v[pl.ds(o, 16)] = jnp.full((16,), 1.0, jnp.float32)
            return carry

        lax.fori_loop(_i32(0), _i32(nvec // _UN), fo, _i32(0))

        def in_start(i, b):
            base = pl.multiple_of(tile_base + i * _i32(_C), 8)
            pltpu.async_copy(u_hbm.at[pl.ds(base, _C)], u_b[b], siu[b])
            pltpu.async_copy(v_hbm.at[pl.ds(base, _C)], v_b[b], siv[b])

        def in_wait(b):
            z = pl.ds(_i32(0), _C)
            pltpu.make_async_copy(u_hbm.at[z], u_b[b], siu[b]).wait()
            pltpu.make_async_copy(v_hbm.at[z], v_b[b], siv[b]).wait()

        def sc_wait(b):
            pltpu.make_async_copy(ones_v, row_sh.at[idx_b[b]],
                                  ssc[b]).wait()

        def out_wait(b):
            z = pl.ds(_i32(0), _C)
            pltpu.make_async_copy(vals_b[b], part0_hbm.at[z], so[b]).wait()

        for r in range(2):
            d = c * _i32(2) + _i32(r)
            pltpu.sync_copy(tab_hbm.at[d, pl.ds(s * _i32(seg), seg)],
                            row_sh.at[pl.ds(s * _i32(seg), seg)])
            pltpu.sync_copy(salt_hbm.at[d], salt_v)
            plsc.subcore_barrier()
            salt = salt_v[...]

            def hash_into(b):
                ub, vb, ib = u_b[b], v_b[b], idx_b[b]

                def hb(k, carry):
                    for t in range(_UN):
                        o = pl.multiple_of(
                            k * _i32(16 * _UN) + _i32(t * 16), 16)
                        sl = pl.ds(o, 16)
                        cb = (ub[sl] * _i32(_MUL_U) + vb[sl] * _i32(_MUL_V))
                        ib[sl] = (cb + salt) & mask
                    return carry

                lax.fori_loop(_i32(0), _i32(nvec // _UN), hb, _i32(0))

            def upd_chunk(i, b, start_next):
                if start_next:
                    in_start(i + _i32(1), 1 - b)
                in_wait(b)
                hash_into(b)
                pltpu.async_copy(ones_v, row_sh.at[idx_b[b]], ssc[b],
                                 add=True)

            in_start(_i32(0), 0)

            def upd_pair(gp, carry):
                for b in (0, 1):
                    i = gp * _i32(2) + _i32(b)

                    @pl.when(gp >= _i32(1))
                    def _():
                        sc_wait(b)

                    upd_chunk(i, b, True)
                return carry

            lax.fori_loop(_i32(0), _i32(npair), upd_pair, _i32(0))
            sc_wait(0)
            upd_chunk(_i32(nch - 1), 0, False)
            sc_wait(1)
            sc_wait(0)
            plsc.subcore_barrier()

            def qry_chunk(i, b, start_next):
                if start_next:
                    in_start(i + _i32(1), 1 - b)
                in_wait(b)
                hash_into(b)
                obase = pl.multiple_of(tile_base + i * _i32(_C), 8)
                osl = pl.ds(obase, _C)
                pltpu.async_copy(row_sh.at[idx_b[b]], vals_b[b], sg[b])
                if r == 1:
                    @pl.when(c == _i32(0))
                    def _():
                        pltpu.async_copy(part0_hbm.at[osl], prev_v, sprev)

                    @pl.when(c == _i32(1))
                    def _():
                        pltpu.async_copy(part1_hbm.at[osl], prev_v, sprev)
                pltpu.make_async_copy(row_sh.at[idx_b[b]], vals_b[b],
                                      sg[b]).wait()
                if r == 1:
                    pltpu.make_async_copy(part0_hbm.at[osl], prev_v,
                                          sprev).wait()
                    vb, pb = vals_b[b], prev_v

                    def mb(k, carry):
                        for t in range(_UN):
                            o = pl.multiple_of(
                                k * _i32(16 * _UN) + _i32(t * 16), 16)
                            sl = pl.ds(o, 16)
                            vb[sl] = jnp.minimum(vb[sl], pb[sl])
                        return carry

                    lax.fori_loop(_i32(0), _i32(nvec // _UN), mb, _i32(0))
                @pl.when(c == _i32(0))
                def _():
                    pltpu.async_copy(vals_b[b], part0_hbm.at[osl], so[b])

                @pl.when(c == _i32(1))
                def _():
                    pltpu.async_copy(vals_b[b], part1_hbm.at[osl], so[b])

            in_start(_i32(0), 0)

            def qry_pair(gp, carry):
                for b in (0, 1):
                    i = gp * _i32(2) + _i32(b)

                    @pl.when(gp >= _i32(1))
                    def _():
                        out_wait(b)

                    qry_chunk(i, b, True)
                return carry

            lax.fori_loop(_i32(0), _i32(npair), qry_pair, _i32(0))
            out_wait(0)
            qry_chunk(_i32(nch - 1), 0, False)
            out_wait(1)
            out_wait(0)
            plsc.subcore_barrier()

    return body(u32, v32, table, salts16)


def _min2_kernel(a_ref, b_ref, o_ref):
    o_ref[...] = jnp.minimum(a_ref[...], b_ref[...])


def kernel(u, v, table, salts):
    n = u.shape[0]
    u32 = u.astype(jnp.int32)
    v32 = v.astype(jnp.int32)
    salts16 = jnp.broadcast_to(salts.astype(jnp.int32), (table.shape[0], 16))
    part0, part1 = _sc_sketch(u32, v32, table, salts16)

    blk = 1 << 17
    grid = -(-n // blk)
    out = pl.pallas_call(
        _min2_kernel,
        grid=(grid,),
        in_specs=[
            pl.BlockSpec((blk,), lambda i: (i,)),
            pl.BlockSpec((blk,), lambda i: (i,)),
        ],
        out_specs=pl.BlockSpec((blk,), lambda i: (i,)),
        out_shape=jax.ShapeDtypeStruct((n,), jnp.float32),
    )(part0, part1)
    return out


# speedup vs baseline: 3852.8130x; 1.1041x over previous
import functools

import jax
import jax.numpy as jnp
from jax import lax
from jax.experimental import pallas as pl
from jax.experimental.pallas import tpu as pltpu
from jax.experimental.pallas import tpu_sc as plsc

_C = 2000
_NS = 16
_NC = 2
_UN = 5

_MUL_U = 31337
_MUL_V = -1640531527


def _i32(x):
    return jnp.int32(x)


def _sc_sketch(u32, v32, table, salts16):
    n = u32.shape[0]
    depth, width = table.shape
    mask = _i32(width - 1)
    per_tile = n // _NS
    nch = per_tile // _C
    npair = nch // 2
    seg = width // _NS
    nvec = _C // 16
    mesh = plsc.VectorSubcoreMesh(core_axis_name="c", subcore_axis_name="s")

    @functools.partial(
        pl.kernel,
        out_type=(jax.ShapeDtypeStruct((n,), jnp.float32),
                  jax.ShapeDtypeStruct((n,), jnp.float32)),
        mesh=mesh,
        scratch_types=[
            pltpu.VMEM_SHARED((width,), jnp.float32),
            pltpu.VMEM((_C,), jnp.int32),
            pltpu.VMEM((_C,), jnp.int32),
            pltpu.VMEM((_C,), jnp.int32),
            pltpu.VMEM((_C,), jnp.int32),
            pltpu.VMEM((_C,), jnp.int32),
            pltpu.VMEM((_C,), jnp.int32),
            pltpu.VMEM((_C,), jnp.float32),
            pltpu.VMEM((_C,), jnp.float32),
            pltpu.VMEM((_C,), jnp.float32),
            pltpu.VMEM((_C,), jnp.float32),
            pltpu.VMEM((_C,), jnp.float32),
            pltpu.VMEM((16,), jnp.int32),
            pltpu.SemaphoreType.DMA,
            pltpu.SemaphoreType.DMA,
            pltpu.SemaphoreType.DMA,
            pltpu.SemaphoreType.DMA,
            pltpu.SemaphoreType.DMA,
            pltpu.SemaphoreType.DMA,
            pltpu.SemaphoreType.DMA,
            pltpu.SemaphoreType.DMA,
            pltpu.SemaphoreType.DMA,
            pltpu.SemaphoreType.DMA,
            pltpu.SemaphoreType.DMA,
            pltpu.SemaphoreType.DMA,
        ],
    )
    def body(u_hbm, v_hbm, tab_hbm, salt_hbm, part0_hbm, part1_hbm,
             row_sh, u0, u1, v0, v1, i0, i1, ones_v, g0, g1, p0, p1, salt_v,
             siu0, siu1, siv0, siv1, ssc0, ssc1, sg0, sg1, sprev0, sprev1,
             so0, so1):
        c = lax.axis_index("c")
        s = lax.axis_index("s")
        tile_base = s * _i32(per_tile)
        u_b, v_b, idx_b = (u0, u1), (v0, v1), (i0, i1)
        vals_b, prev_b = (g0, g1), (p0, p1)
        siu, siv = (siu0, siu1), (siv0, siv1)
        ssc, sg, so = (ssc0, ssc1), (sg0, sg1), (so0, so1)
        sprev = (sprev0, sprev1)

        def fo(k, carry):
            for t in range(_UN):
                o = pl.multiple_of(k * _i32(16 * _UN) + _i32(t * 16), 16)
                ones_v[pl.ds(o, 16)] = jnp.full((16,), 1.0, jnp.float32)
            return carry

        lax.fori_loop(_i32(0), _i32(nvec // _UN), fo, _i32(0))

        def in_start(i, b):
            base = pl.multiple_of(tile_base + i * _i32(_C), 8)
            pltpu.async_copy(u_hbm.at[pl.ds(base, _C)], u_b[b], siu[b])
            pltpu.async_copy(v_hbm.at[pl.ds(base, _C)], v_b[b], siv[b])

        def in_wait(b):
            z = pl.ds(_i32(0), _C)
            pltpu.make_async_copy(u_hbm.at[z], u_b[b], siu[b]).wait()
            pltpu.make_async_copy(v_hbm.at[z], v_b[b], siv[b]).wait()

        def sc_wait(b):
            pltpu.make_async_copy(ones_v, row_sh.at[idx_b[b]],
                                  ssc[b]).wait()

        def out_wait(b):
            z = pl.ds(_i32(0), _C)
            pltpu.make_async_copy(vals_b[b], part0_hbm.at[z], so[b]).wait()

        for r in range(2):
            d = c * _i32(2) + _i32(r)
            pltpu.sync_copy(tab_hbm.at[d, pl.ds(s * _i32(seg), seg)],
                            row_sh.at[pl.ds(s * _i32(seg), seg)])
            pltpu.sync_copy(salt_hbm.at[d], salt_v)
            plsc.subcore_barrier()
            salt = salt_v[...]

            def hash_into(b):
                ub, vb, ib = u_b[b], v_b[b], idx_b[b]

                def hb(k, carry):
                    for t in range(_UN):
                        o = pl.multiple_of(
                            k * _i32(16 * _UN) + _i32(t * 16), 16)
                        sl = pl.ds(o, 16)
                        cb = (ub[sl] * _i32(_MUL_U) + vb[sl] * _i32(_MUL_V))
                        ib[sl] = (cb + salt) & mask
                    return carry

                lax.fori_loop(_i32(0), _i32(nvec // _UN), hb, _i32(0))

            def upd_chunk(i, b, start_next):
                if start_next:
                    in_start(i + _i32(1), 1 - b)
                in_wait(b)
                hash_into(b)
                pltpu.async_copy(ones_v, row_sh.at[idx_b[b]], ssc[b],
                                 add=True)

            in_start(_i32(0), 0)

            def upd_pair(gp, carry):
                for b in (0, 1):
                    i = gp * _i32(2) + _i32(b)

                    @pl.when(gp >= _i32(1))
                    def _():
                        sc_wait(b)

                    upd_chunk(i, b, True)
                return carry

            lax.fori_loop(_i32(0), _i32(npair), upd_pair, _i32(0))
            sc_wait(0)
            upd_chunk(_i32(nch - 1), 0, False)
            sc_wait(1)
            sc_wait(0)
            plsc.subcore_barrier()

            def qry_issue(i, b, nxt, ow):
                if nxt is True:
                    in_start(i + _i32(1), 1 - b)
                elif nxt is not None:
                    @pl.when(nxt)
                    def _():
                        in_start(i + _i32(1), 1 - b)
                in_wait(b)
                hash_into(b)
                if ow is True:
                    out_wait(b)
                elif ow is not None:
                    @pl.when(ow)
                    def _():
                        out_wait(b)
                pltpu.async_copy(row_sh.at[idx_b[b]], vals_b[b], sg[b])
                if r == 1:
                    obase = pl.multiple_of(tile_base + i * _i32(_C), 8)
                    osl = pl.ds(obase, _C)

                    @pl.when(c == _i32(0))
                    def _():
                        pltpu.async_copy(part0_hbm.at[osl], prev_b[b],
                                         sprev[b])

                    @pl.when(c == _i32(1))
                    def _():
                        pltpu.async_copy(part1_hbm.at[osl], prev_b[b],
                                         sprev[b])

            def qry_finish(i, b):
                obase = pl.multiple_of(tile_base + i * _i32(_C), 8)
                osl = pl.ds(obase, _C)
                pltpu.make_async_copy(row_sh.at[idx_b[b]], vals_b[b],
                                      sg[b]).wait()
                if r == 1:
                    pltpu.make_async_copy(part0_hbm.at[osl], prev_b[b],
                                          sprev[b]).wait()
                    vb, pb = vals_b[b], prev_b[b]

                    def mb(k, carry):
                        for t in range(_UN):
                            o = pl.multiple_of(
                                k * _i32(16 * _UN) + _i32(t * 16), 16)
                            sl = pl.ds(o, 16)
                            vb[sl] = jnp.minimum(vb[sl], pb[sl])
                        return carry

                    lax.fori_loop(_i32(0), _i32(nvec // _UN), mb, _i32(0))
                @pl.when(c == _i32(0))
                def _():
                    pltpu.async_copy(vals_b[b], part0_hbm.at[osl], so[b])

                @pl.when(c == _i32(1))
                def _():
                    pltpu.async_copy(vals_b[b], part1_hbm.at[osl], so[b])

            in_start(_i32(0), 0)
            qry_issue(_i32(0), 0, True, None)

            def qry_pair(gp, carry):
                qry_issue(gp * _i32(2) + _i32(1), 1, True, gp >= _i32(1))
                qry_finish(gp * _i32(2), 0)
                qry_issue(gp * _i32(2) + _i32(2), 0,
                          gp < _i32(npair - 1), True)
                qry_finish(gp * _i32(2) + _i32(1), 1)
                return carry

            lax.fori_loop(_i32(0), _i32(npair), qry_pair, _i32(0))
            qry_finish(_i32(nch - 1), 0)
            out_wait(1)
            out_wait(0)
            plsc.subcore_barrier()

    return body(u32, v32, table, salts16)


def _min2_kernel(a_ref, b_ref, o_ref):
    o_ref[...] = jnp.minimum(a_ref[...], b_ref[...])


def kernel(u, v, table, salts):
    n = u.shape[0]
    u32 = u.astype(jnp.int32)
    v32 = v.astype(jnp.int32)
    salts16 = jnp.broadcast_to(salts.astype(jnp.int32), (table.shape[0], 16))
    part0, part1 = _sc_sketch(u32, v32, table, salts16)

    blk = 1 << 17
    grid = -(-n // blk)
    out = pl.pallas_call(
        _min2_kernel,
        grid=(grid,),
        in_specs=[
            pl.BlockSpec((blk,), lambda i: (i,)),
            pl.BlockSpec((blk,), lambda i: (i,)),
        ],
        out_specs=pl.BlockSpec((blk,), lambda i: (i,)),
        out_shape=jax.ShapeDtypeStruct((n,), jnp.float32),
    )(part0, part1)
    return out


# speedup vs baseline: 3854.9542x; 1.0006x over previous
import functools

import jax
import jax.numpy as jnp
from jax import lax
from jax.experimental import pallas as pl
from jax.experimental.pallas import tpu as pltpu
from jax.experimental.pallas import tpu_sc as plsc

_C = 2000
_NS = 16
_NC = 2
_UN = 5

_MUL_U = 31337
_MUL_V = -1640531527


def _i32(x):
    return jnp.int32(x)


def _sc_sketch(u32, v32, table, salts16):
    n = u32.shape[0]
    depth, width = table.shape
    mask = _i32(width - 1)
    per_tile = n // _NS
    nch = per_tile // _C
    npair = nch // 2
    seg = width // _NS
    nvec = _C // 16
    mesh = plsc.VectorSubcoreMesh(core_axis_name="c", subcore_axis_name="s")

    @functools.partial(
        pl.kernel,
        out_type=(jax.ShapeDtypeStruct((n,), jnp.float32),
                  jax.ShapeDtypeStruct((n,), jnp.float32)),
        mesh=mesh,
        scratch_types=[
            pltpu.VMEM_SHARED((width,), jnp.float32),
            pltpu.VMEM((_C,), jnp.int32),
            pltpu.VMEM((_C,), jnp.int32),
            pltpu.VMEM((_C,), jnp.int32),
            pltpu.VMEM((_C,), jnp.int32),
            pltpu.VMEM((_C,), jnp.int32),
            pltpu.VMEM((_C,), jnp.int32),
            pltpu.VMEM((_C,), jnp.float32),
            pltpu.VMEM((_C,), jnp.float32),
            pltpu.VMEM((_C,), jnp.float32),
            pltpu.VMEM((_C,), jnp.float32),
            pltpu.VMEM((_C,), jnp.float32),
            pltpu.VMEM((16,), jnp.int32),
            pltpu.SemaphoreType.DMA,
            pltpu.SemaphoreType.DMA,
            pltpu.SemaphoreType.DMA,
            pltpu.SemaphoreType.DMA,
            pltpu.SemaphoreType.DMA,
            pltpu.SemaphoreType.DMA,
            pltpu.SemaphoreType.DMA,
            pltpu.SemaphoreType.DMA,
            pltpu.SemaphoreType.DMA,
            pltpu.SemaphoreType.DMA,
            pltpu.SemaphoreType.DMA,
            pltpu.SemaphoreType.DMA,
        ],
    )
    def body(u_hbm, v_hbm, tab_hbm, salt_hbm, part0_hbm, part1_hbm,
             row_sh, u0, u1, v0, v1, i0, i1, ones_v, g0, g1, p0, p1, salt_v,
             siu0, siu1, siv0, siv1, ssc0, ssc1, sg0, sg1, sprev0, sprev1,
             so0, so1):
        c = lax.axis_index("c")
        s = lax.axis_index("s")
        tile_base = s * _i32(per_tile)
        u_b, v_b, idx_b = (u0, u1), (v0, v1), (i0, i1)
        vals_b, prev_b = (g0, g1), (p0, p1)
        siu, siv = (siu0, siu1), (siv0, siv1)
        ssc, sg, so = (ssc0, ssc1), (sg0, sg1), (so0, so1)
        sprev = (sprev0, sprev1)

        def fo(k, carry):
            for t in range(_UN):
                o = pl.multiple_of(k * _i32(16 * _UN) + _i32(t * 16), 16)
                ones_v[pl.ds(o, 16)] = jnp.full((16,), 1.0, jnp.float32)
            return carry

        lax.fori_loop(_i32(0), _i32(nvec // _UN), fo, _i32(0))

        def in_start(i, b):
            base = pl.multiple_of(tile_base + i * _i32(_C), 8)
            pltpu.async_copy(u_hbm.at[pl.ds(base, _C)], u_b[b], siu[b])
            pltpu.async_copy(v_hbm.at[pl.ds(base, _C)], v_b[b], siv[b])

        def in_wait(b):
            z = pl.ds(_i32(0), _C)
            pltpu.make_async_copy(u_hbm.at[z], u_b[b], siu[b]).wait()
            pltpu.make_async_copy(v_hbm.at[z], v_b[b], siv[b]).wait()

        def sc_wait(b):
            pltpu.make_async_copy(ones_v, row_sh.at[idx_b[b]],
                                  ssc[b]).wait()

        def out_wait(b):
            z = pl.ds(_i32(0), _C)
            pltpu.make_async_copy(vals_b[b], part0_hbm.at[z], so[b]).wait()

        for r in range(2):
            d = c * _i32(2) + _i32(r)
            pltpu.sync_copy(tab_hbm.at[d, pl.ds(s * _i32(seg), seg)],
                            row_sh.at[pl.ds(s * _i32(seg), seg)])
            pltpu.sync_copy(salt_hbm.at[d], salt_v)
            plsc.subcore_barrier()
            salt = salt_v[...]

            def hash_into(b):
                ub, vb, ib = u_b[b], v_b[b], idx_b[b]

                def hb(k, carry):
                    for t in range(_UN):
                        o = pl.multiple_of(
                            k * _i32(16 * _UN) + _i32(t * 16), 16)
                        sl = pl.ds(o, 16)
                        cb = (ub[sl] * _i32(_MUL_U) + vb[sl] * _i32(_MUL_V))
                        ib[sl] = (cb + salt) & mask
                    return carry

                lax.fori_loop(_i32(0), _i32(nvec // _UN), hb, _i32(0))

            def upd_chunk(i, b, start_next):
                if start_next:
                    in_start(i + _i32(1), 1 - b)
                in_wait(b)
                hash_into(b)
                pltpu.async_copy(ones_v, row_sh.at[idx_b[b]], ssc[b],
                                 add=True)

            in_start(_i32(0), 0)

            def upd_pair(gp, carry):
                for b in (0, 1):
                    i = gp * _i32(2) + _i32(b)

                    @pl.when(gp >= _i32(1))
                    def _():
                        sc_wait(b)

                    upd_chunk(i, b, True)
                return carry

            lax.fori_loop(_i32(0), _i32(npair), upd_pair, _i32(0))
            sc_wait(0)
            upd_chunk(_i32(nch - 1), 0, False)
            sc_wait(1)
            sc_wait(0)
            plsc.subcore_barrier()

            def qry_issue(i, b, nxt, ow):
                if nxt is True:
                    in_start(i + _i32(1), 1 - b)
                elif nxt is not None:
                    @pl.when(nxt)
                    def _():
                        in_start(i + _i32(1), 1 - b)
                in_wait(b)
                hash_into(b)
                if ow is True:
                    out_wait(b)
                elif ow is not None:
                    @pl.when(ow)
                    def _():
                        out_wait(b)
                if r == 1:
                    obase = pl.multiple_of(tile_base + i * _i32(_C), 8)
                    osl = pl.ds(obase, _C)

                    @pl.when(c == _i32(0))
                    def _():
                        pltpu.async_copy(part0_hbm.at[osl], prev_b[b],
                                         sprev[b])

                    @pl.when(c == _i32(1))
                    def _():
                        pltpu.async_copy(part1_hbm.at[osl], prev_b[b],
                                         sprev[b])
                pltpu.async_copy(row_sh.at[idx_b[b]], vals_b[b], sg[b])

            def qry_finish(i, b):
                obase = pl.multiple_of(tile_base + i * _i32(_C), 8)
                osl = pl.ds(obase, _C)
                pltpu.make_async_copy(row_sh.at[idx_b[b]], vals_b[b],
                                      sg[b]).wait()
                if r == 1:
                    pltpu.make_async_copy(part0_hbm.at[osl], prev_b[b],
                                          sprev[b]).wait()
                    vb, pb = vals_b[b], prev_b[b]

                    def mb(k, carry):
                        for t in range(_UN):
                            o = pl.multiple_of(
                                k * _i32(16 * _UN) + _i32(t * 16), 16)
                            sl = pl.ds(o, 16)
                            vb[sl] = jnp.minimum(vb[sl], pb[sl])
                        return carry

                    lax.fori_loop(_i32(0), _i32(nvec // _UN), mb, _i32(0))
                @pl.when(c == _i32(0))
                def _():
                    pltpu.async_copy(vals_b[b], part0_hbm.at[osl], so[b])

                @pl.when(c == _i32(1))
                def _():
                    pltpu.async_copy(vals_b[b], part1_hbm.at[osl], so[b])

            in_start(_i32(0), 0)
            qry_issue(_i32(0), 0, True, None)

            def qry_pair(gp, carry):
                qry_issue(gp * _i32(2) + _i32(1), 1, True, gp >= _i32(1))
                qry_finish(gp * _i32(2), 0)
                qry_issue(gp * _i32(2) + _i32(2), 0,
                          gp < _i32(npair - 1), True)
                qry_finish(gp * _i32(2) + _i32(1), 1)
                return carry

            lax.fori_loop(_i32(0), _i32(npair), qry_pair, _i32(0))
            qry_finish(_i32(nch - 1), 0)
            out_wait(1)
            out_wait(0)
            plsc.subcore_barrier()

    return body(u32, v32, table, salts16)


def _min2_kernel(a_ref, b_ref, o_ref):
    o_ref[...] = jnp.minimum(a_ref[...], b_ref[...])


def kernel(u, v, table, salts):
    n = u.shape[0]
    u32 = u.astype(jnp.int32)
    v32 = v.astype(jnp.int32)
    salts16 = jnp.broadcast_to(salts.astype(jnp.int32), (table.shape[0], 16))
    part0, part1 = _sc_sketch(u32, v32, table, salts16)

    blk = 1 << 17
    grid = -(-n // blk)
    out = pl.pallas_call(
        _min2_kernel,
        grid=(grid,),
        in_specs=[
            pl.BlockSpec((blk,), lambda i: (i,)),
            pl.BlockSpec((blk,), lambda i: (i,)),
        ],
        out_specs=pl.BlockSpec((blk,), lambda i: (i,)),
        out_shape=jax.ShapeDtypeStruct((n,), jnp.float32),
    )(part0, part1)
    return out
